# baseline (device time: 71541 ns/iter reference)
import jax
import jax.numpy as jnp
from jax import lax
from jax.experimental import pallas as pl
from jax.experimental.pallas import tpu as pltpu

N_DEV = 4


def kernel(x, Win0, Wout0, Win1, Wout1, Win2, Wout2):
    m, d = x.shape

    def body(x_ref, win0_ref, wout0_ref, win1_ref, wout1_ref,
             win2_ref, wout2_ref, out_ref,
             xfull_ref, part_ref, acc_ref, rcv_ref,
             ag_send, ag_recv, rs_send, rs_recv):
        my = lax.axis_index("i")
        left = lax.rem(my + N_DEV - 1, N_DEV)
        right = lax.rem(my + 1, N_DEV)

        barrier = pltpu.get_barrier_semaphore()
        for nbr in (left, right):
            pl.semaphore_signal(barrier, inc=1, device_id=(nbr,),
                                device_id_type=pl.DeviceIdType.MESH)
        pl.semaphore_wait(barrier, 2)

        xfull_ref[my] = x_ref[...]

        wins = [win0_ref, win1_ref, win2_ref]
        wouts = [wout0_ref, wout1_ref, wout2_ref]

        for layer in range(3):
            for hop in range(N_DEV - 1):
                c = lax.rem(my + N_DEV - hop, N_DEV)
                rdma = pltpu.make_async_remote_copy(
                    src_ref=xfull_ref.at[c],
                    dst_ref=xfull_ref.at[c],
                    send_sem=ag_send.at[hop],
                    recv_sem=ag_recv.at[hop],
                    device_id=(right,),
                    device_id_type=pl.DeviceIdType.MESH,
                )
                rdma.start()
                rdma.wait()

            x2d = xfull_ref[...].reshape(N_DEV * m, d)
            hid = jnp.maximum(
                jnp.dot(x2d, wins[layer][...],
                        preferred_element_type=jnp.float32), 0.0)
            part = jnp.dot(hid, wouts[layer][...],
                           preferred_element_type=jnp.float32)
            part_ref[...] = part.reshape(N_DEV, m, d)

            for s in range(N_DEV - 1):
                csend = lax.rem(my + N_DEV - 1 - s, N_DEV)
                src = part_ref.at[csend] if s == 0 else acc_ref.at[s - 1]
                rdma = pltpu.make_async_remote_copy(
                    src_ref=src,
                    dst_ref=rcv_ref.at[s],
                    send_sem=rs_send.at[s],
                    recv_sem=rs_recv.at[s],
                    device_id=(right,),
                    device_id_type=pl.DeviceIdType.MESH,
                )
                rdma.start()
                rdma.wait()
                crecv = lax.rem(my + N_DEV - 2 - s, N_DEV)
                if s < N_DEV - 2:
                    acc_ref[s] = rcv_ref[s] + part_ref[crecv]
                else:
                    y = rcv_ref[s] + part_ref[my]
                    if layer < 2:
                        xfull_ref[my] = y
                    else:
                        out_ref[...] = y

    return pl.pallas_call(
        body,
        out_shape=jax.ShapeDtypeStruct((m, d), jnp.float32),
        in_specs=[pl.BlockSpec(memory_space=pltpu.VMEM)] * 7,
        out_specs=pl.BlockSpec(memory_space=pltpu.VMEM),
        scratch_shapes=[
            pltpu.VMEM((N_DEV, m, d), jnp.float32),
            pltpu.VMEM((N_DEV, m, d), jnp.float32),
            pltpu.VMEM((N_DEV - 2, m, d), jnp.float32),
            pltpu.VMEM((N_DEV - 1, m, d), jnp.float32),
            pltpu.SemaphoreType.DMA((N_DEV - 1,)),
            pltpu.SemaphoreType.DMA((N_DEV - 1,)),
            pltpu.SemaphoreType.DMA((N_DEV - 1,)),
            pltpu.SemaphoreType.DMA((N_DEV - 1,)),
        ],
        compiler_params=pltpu.CompilerParams(collective_id=0),
    )(x, Win0, Wout0, Win1, Wout1, Win2, Wout2)


# device time: 54196 ns/iter; 1.3200x vs baseline; 1.3200x over previous
import jax
import jax.numpy as jnp
from jax import lax
from jax.experimental import pallas as pl
from jax.experimental.pallas import tpu as pltpu

N_DEV = 4


def kernel(x, Win0, Wout0, Win1, Wout1, Win2, Wout2):
    m, d = x.shape

    def body(x_ref, win0_ref, wout0_ref, win1_ref, wout1_ref,
             win2_ref, wout2_ref, out_ref,
             xfull_ref, part_ref, acc_ref, rcv_ref,
             ag_send, ag_recv, rs_send, rs_recv):
        my = lax.axis_index("i")
        left = lax.rem(my + N_DEV - 1, N_DEV)
        right = lax.rem(my + 1, N_DEV)
        c_m1 = lax.rem(my + 3, N_DEV)
        c_m2 = lax.rem(my + 2, N_DEV)
        c_p1 = lax.rem(my + 1, N_DEV)

        barrier = pltpu.get_barrier_semaphore()
        for nbr in (left, right):
            pl.semaphore_signal(barrier, inc=1, device_id=(nbr,),
                                device_id_type=pl.DeviceIdType.MESH)
        pl.semaphore_wait(barrier, 2)

        xfull_ref[my] = x_ref[...]

        wins = [win0_ref, win1_ref, win2_ref]
        wouts = [wout0_ref, wout1_ref, wout2_ref]

        def make_ag(hop):
            c = lax.rem(my + N_DEV - hop, N_DEV)
            return pltpu.make_async_remote_copy(
                src_ref=xfull_ref.at[c], dst_ref=xfull_ref.at[c],
                send_sem=ag_send.at[hop], recv_sem=ag_recv.at[hop],
                device_id=(right,), device_id_type=pl.DeviceIdType.MESH)

        def make_rs(s):
            csend = lax.rem(my + N_DEV - 1 - s, N_DEV)
            src = part_ref.at[csend] if s == 0 else acc_ref.at[s - 1]
            return pltpu.make_async_remote_copy(
                src_ref=src, dst_ref=rcv_ref.at[s],
                send_sem=rs_send.at[s], recv_sem=rs_recv.at[s],
                device_id=(right,), device_id_type=pl.DeviceIdType.MESH)

        for layer in range(3):
            win, wout = wins[layer], wouts[layer]

            def compute_chunk(c):
                xc = xfull_ref[c]
                hc = jnp.maximum(
                    jnp.dot(xc, win[...],
                            preferred_element_type=jnp.float32), 0.0)
                part_ref[c] = jnp.dot(
                    hc, wout[...], preferred_element_type=jnp.float32)

            ag0 = make_ag(0)
            ag0.start()
            compute_chunk(my)
            ag0.wait()
            ag1 = make_ag(1)
            ag1.start()
            compute_chunk(c_m1)
            rs0 = make_rs(0)
            rs0.start()
            ag1.wait()
            ag2 = make_ag(2)
            ag2.start()
            compute_chunk(c_m2)
            rs0.wait()
            acc_ref[0] = rcv_ref[0] + part_ref[c_m2]
            rs1 = make_rs(1)
            rs1.start()
            ag2.wait()
            compute_chunk(c_p1)
            rs1.wait()
            acc_ref[1] = rcv_ref[1] + part_ref[c_p1]
            rs2 = make_rs(2)
            rs2.start()
            rs2.wait()
            y = rcv_ref[2] + part_ref[my]
            if layer < 2:
                xfull_ref[my] = y
            else:
                out_ref[...] = y

    return pl.pallas_call(
        body,
        out_shape=jax.ShapeDtypeStruct((m, d), jnp.float32),
        in_specs=[pl.BlockSpec(memory_space=pltpu.VMEM)] * 7,
        out_specs=pl.BlockSpec(memory_space=pltpu.VMEM),
        scratch_shapes=[
            pltpu.VMEM((N_DEV, m, d), jnp.float32),
            pltpu.VMEM((N_DEV, m, d), jnp.float32),
            pltpu.VMEM((N_DEV - 2, m, d), jnp.float32),
            pltpu.VMEM((N_DEV - 1, m, d), jnp.float32),
            pltpu.SemaphoreType.DMA((N_DEV - 1,)),
            pltpu.SemaphoreType.DMA((N_DEV - 1,)),
            pltpu.SemaphoreType.DMA((N_DEV - 1,)),
            pltpu.SemaphoreType.DMA((N_DEV - 1,)),
        ],
        compiler_params=pltpu.CompilerParams(collective_id=0),
    )(x, Win0, Wout0, Win1, Wout1, Win2, Wout2)


# device time: 41868 ns/iter; 1.7087x vs baseline; 1.2944x over previous
import jax
import jax.numpy as jnp
from jax import lax
from jax.experimental import pallas as pl
from jax.experimental.pallas import tpu as pltpu

N_DEV = 4


def kernel(x, Win0, Wout0, Win1, Wout1, Win2, Wout2):
    m, d = x.shape

    def body(x_ref, win0_ref, wout0_ref, win1_ref, wout1_ref,
             win2_ref, wout2_ref, out_ref,
             xfull_ref, part_ref, rcv_ref,
             ag_send, ag_recv, rs_send, rs_recv):
        my = lax.axis_index("i")
        left = lax.rem(my + 3, N_DEV)
        right = lax.rem(my + 1, N_DEV)
        diag = lax.rem(my + 2, N_DEV)

        def copy(src, dst, ssem, rsem, dev):
            return pltpu.make_async_remote_copy(
                src_ref=src, dst_ref=dst, send_sem=ssem, recv_sem=rsem,
                device_id=(dev,), device_id_type=pl.DeviceIdType.MESH)

        barrier = pltpu.get_barrier_semaphore()
        for nbr in (left, right, diag):
            pl.semaphore_signal(barrier, inc=1, device_id=(nbr,),
                                device_id_type=pl.DeviceIdType.MESH)
        pl.semaphore_wait(barrier, 3)

        xfull_ref[my] = x_ref[...]

        wins = [win0_ref, win1_ref, win2_ref]
        wouts = [wout0_ref, wout1_ref, wout2_ref]
        rs_sends = []

        for layer in range(3):
            win, wout = wins[layer], wouts[layer]

            def compute_chunk(c):
                hc = jnp.maximum(
                    jnp.dot(xfull_ref[c], win[...],
                            preferred_element_type=jnp.float32), 0.0)
                part_ref[c] = jnp.dot(
                    hc, wout[...], preferred_element_type=jnp.float32)

            ag_to_r = copy(xfull_ref.at[my], xfull_ref.at[my],
                           ag_send.at[0], ag_recv.at[0], right)
            ag_to_l = copy(xfull_ref.at[my], xfull_ref.at[my],
                           ag_send.at[1], ag_recv.at[1], left)
            ag_to_d = copy(xfull_ref.at[my], xfull_ref.at[my],
                           ag_send.at[2], ag_recv.at[2], diag)
            ag_to_r.start()
            ag_to_l.start()
            ag_to_d.start()
            for s in rs_sends:
                s.wait_send()

            compute_chunk(my)

            ag_fr_l = copy(xfull_ref.at[left], xfull_ref.at[left],
                           ag_send.at[0], ag_recv.at[0], left)
            ag_fr_r = copy(xfull_ref.at[right], xfull_ref.at[right],
                           ag_send.at[1], ag_recv.at[1], right)
            ag_fr_d = copy(xfull_ref.at[diag], xfull_ref.at[diag],
                           ag_send.at[2], ag_recv.at[2], diag)

            ag_fr_l.wait_recv()
            compute_chunk(left)
            rs_to_l = copy(part_ref.at[left], rcv_ref.at[1],
                           rs_send.at[1], rs_recv.at[1], left)
            rs_to_l.start()
            ag_fr_r.wait_recv()
            compute_chunk(right)
            rs_to_r = copy(part_ref.at[right], rcv_ref.at[0],
                           rs_send.at[0], rs_recv.at[0], right)
            rs_to_r.start()
            ag_fr_d.wait_recv()
            compute_chunk(diag)
            rs_to_d = copy(part_ref.at[diag], rcv_ref.at[2],
                           rs_send.at[2], rs_recv.at[2], diag)
            rs_to_d.start()
            rs_sends = [rs_to_l, rs_to_r, rs_to_d]

            ag_to_r.wait_send()
            ag_to_l.wait_send()
            ag_to_d.wait_send()

            rs_fr_l = copy(part_ref.at[left], rcv_ref.at[0],
                           rs_send.at[0], rs_recv.at[0], left)
            rs_fr_r = copy(part_ref.at[right], rcv_ref.at[1],
                           rs_send.at[1], rs_recv.at[1], right)
            rs_fr_d = copy(part_ref.at[diag], rcv_ref.at[2],
                           rs_send.at[2], rs_recv.at[2], diag)
            rs_fr_l.wait_recv()
            rs_fr_r.wait_recv()
            rs_fr_d.wait_recv()
            y = (part_ref[my] + rcv_ref[0]) + (rcv_ref[1] + rcv_ref[2])
            if layer < 2:
                xfull_ref[my] = y
            else:
                out_ref[...] = y
                for s in rs_sends:
                    s.wait_send()

    return pl.pallas_call(
        body,
        out_shape=jax.ShapeDtypeStruct((m, d), jnp.float32),
        in_specs=[pl.BlockSpec(memory_space=pltpu.VMEM)] * 7,
        out_specs=pl.BlockSpec(memory_space=pltpu.VMEM),
        scratch_shapes=[
            pltpu.VMEM((N_DEV, m, d), jnp.float32),
            pltpu.VMEM((N_DEV, m, d), jnp.float32),
            pltpu.VMEM((N_DEV - 1, m, d), jnp.float32),
            pltpu.SemaphoreType.DMA((N_DEV - 1,)),
            pltpu.SemaphoreType.DMA((N_DEV - 1,)),
            pltpu.SemaphoreType.DMA((N_DEV - 1,)),
            pltpu.SemaphoreType.DMA((N_DEV - 1,)),
        ],
        compiler_params=pltpu.CompilerParams(collective_id=0),
    )(x, Win0, Wout0, Win1, Wout1, Win2, Wout2)


# device time: 40351 ns/iter; 1.7730x vs baseline; 1.0376x over previous
import jax
import jax.numpy as jnp
from jax import lax
from jax.experimental import pallas as pl
from jax.experimental.pallas import tpu as pltpu

N_DEV = 4
R = [0, 2, 3, 0]


def kernel(x, Win0, Wout0, Win1, Wout1, Win2, Wout2):
    m, d = x.shape

    def body(x_ref, win0_ref, wout0_ref, win1_ref, wout1_ref,
             win2_ref, wout2_ref, out_ref,
             xbufA_ref, xbufB_ref, part_ref, rcv_ref,
             ag_send, ag_recv, rs_send, rs_recv):
        my = lax.axis_index("i")
        left = lax.rem(my + 3, N_DEV)
        right = lax.rem(my + 1, N_DEV)
        diag = lax.rem(my + 2, N_DEV)

        def cix(off):
            return lax.rem(my + (off % N_DEV), N_DEV)

        def copy(src, dst, ssem, rsem, dev):
            return pltpu.make_async_remote_copy(
                src_ref=src, dst_ref=dst, send_sem=ssem, recv_sem=rsem,
                device_id=(dev,), device_id_type=pl.DeviceIdType.MESH)

        barrier = pltpu.get_barrier_semaphore()
        for nbr in (left, right, diag):
            pl.semaphore_signal(barrier, inc=1, device_id=(nbr,),
                                device_id_type=pl.DeviceIdType.MESH)
        pl.semaphore_wait(barrier, 3)

        xbufA_ref[my] = x_ref[...]

        wins = [win0_ref, win1_ref, win2_ref]
        wouts = [wout0_ref, wout1_ref, wout2_ref]
        rs_sends = []

        for layer in range(3):
            win, wout = wins[layer], wouts[layer]
            xbuf = xbufA_ref if layer % 2 == 0 else xbufB_ref
            xnext = xbufB_ref if layer % 2 == 0 else xbufA_ref
            r, rp = R[layer], R[layer + 1]
            delta = (rp - r) % N_DEV
            c_own = cix(-r)
            c_l = cix(-1 - r)
            c_r = cix(1 - r)
            c_d = cix(2 - r)
            c_keep = cix(-rp)

            def compute_chunk(c):
                hc = jnp.maximum(
                    jnp.dot(xbuf[c], win[...],
                            preferred_element_type=jnp.float32), 0.0)
                part_ref[c] = jnp.dot(
                    hc, wout[...], preferred_element_type=jnp.float32)

            ag_to_r = copy(xbuf.at[c_own], xbuf.at[c_own],
                           ag_send.at[0], ag_recv.at[0], right)
            ag_to_l = copy(xbuf.at[c_own], xbuf.at[c_own],
                           ag_send.at[1], ag_recv.at[1], left)
            ag_to_d = copy(xbuf.at[c_own], xbuf.at[c_own],
                           ag_send.at[2], ag_recv.at[2], diag)
            ag_to_r.start()
            ag_to_l.start()
            ag_to_d.start()
            for s in rs_sends:
                s.wait_send()

            ag_fr_l = copy(xbuf.at[c_l], xbuf.at[c_l],
                           ag_send.at[0], ag_recv.at[0], left)
            ag_fr_r = copy(xbuf.at[c_r], xbuf.at[c_r],
                           ag_send.at[1], ag_recv.at[1], right)
            ag_fr_d = copy(xbuf.at[c_d], xbuf.at[c_d],
                           ag_send.at[2], ag_recv.at[2], diag)

            def rs_to(dev, slot, c):
                s = copy(part_ref.at[c], rcv_ref.at[slot],
                         rs_send.at[slot], rs_recv.at[slot], dev)
                s.start()
                return s

            if delta == 2:
                compute_chunk(c_own)
                s_d = rs_to(diag, 2, c_own)
                ag_fr_l.wait_recv()
                compute_chunk(c_l)
                s_r = rs_to(right, 0, c_l)
                ag_fr_r.wait_recv()
                compute_chunk(c_r)
                s_l = rs_to(left, 1, c_r)
                ag_fr_d.wait_recv()
                compute_chunk(c_d)
            else:
                compute_chunk(c_own)
                s_r = rs_to(right, 0, c_own)
                ag_fr_r.wait_recv()
                compute_chunk(c_r)
                s_d = rs_to(diag, 2, c_r)
                ag_fr_l.wait_recv()
                compute_chunk(c_l)
                ag_fr_d.wait_recv()
                compute_chunk(c_d)
                s_l = rs_to(left, 1, c_d)
            rs_sends = [s_l, s_r, s_d]

            ag_to_r.wait_send()
            ag_to_l.wait_send()
            ag_to_d.wait_send()

            rs_fr_l = copy(part_ref.at[c_keep], rcv_ref.at[0],
                           rs_send.at[0], rs_recv.at[0], left)
            rs_fr_r = copy(part_ref.at[c_keep], rcv_ref.at[1],
                           rs_send.at[1], rs_recv.at[1], right)
            rs_fr_d = copy(part_ref.at[c_keep], rcv_ref.at[2],
                           rs_send.at[2], rs_recv.at[2], diag)
            rs_fr_l.wait_recv()
            rs_fr_r.wait_recv()
            rs_fr_d.wait_recv()
            y = (part_ref[c_keep] + rcv_ref[0]) + (rcv_ref[1] + rcv_ref[2])
            if layer < 2:
                xnext[c_keep] = y
            else:
                out_ref[...] = y
                for s in rs_sends:
                    s.wait_send()

    return pl.pallas_call(
        body,
        out_shape=jax.ShapeDtypeStruct((m, d), jnp.float32),
        in_specs=[pl.BlockSpec(memory_space=pltpu.VMEM)] * 7,
        out_specs=pl.BlockSpec(memory_space=pltpu.VMEM),
        scratch_shapes=[
            pltpu.VMEM((N_DEV, m, d), jnp.float32),
            pltpu.VMEM((N_DEV, m, d), jnp.float32),
            pltpu.VMEM((N_DEV, m, d), jnp.float32),
            pltpu.VMEM((N_DEV - 1, m, d), jnp.float32),
            pltpu.SemaphoreType.DMA((N_DEV - 1,)),
            pltpu.SemaphoreType.DMA((N_DEV - 1,)),
            pltpu.SemaphoreType.DMA((N_DEV - 1,)),
            pltpu.SemaphoreType.DMA((N_DEV - 1,)),
        ],
        compiler_params=pltpu.CompilerParams(collective_id=0),
    )(x, Win0, Wout0, Win1, Wout1, Win2, Wout2)
